# baseline (speedup 1.0000x reference)
import functools
import math

import jax
import jax.numpy as jnp
from jax import lax
from jax.experimental import pallas as pl
from jax.experimental.pallas import tpu as pltpu
from jax.experimental.pallas import tpu_sc as plsc

_MARGIN = 0.5
_SCALE = 64.0
_COS_M = math.cos(_MARGIN)
_SIN_M = math.sin(_MARGIN)
_OVERFLOW_THRESH = -math.cos(_MARGIN)
_C1 = _SCALE * math.log2(math.e)
_LN2 = math.log(2.0)


def _margined(t):
    tm = t * _COS_M - jnp.sqrt(jnp.maximum(1.0 - t * t, 0.0)) * _SIN_M
    return jnp.where(t < _OVERFLOW_THRESH, t, tm)


def _sc_gather_body(nclass, chunk, flat_ref, y_ref, t_ref, y_v, idx_v, t_v, sem):
    wid = lax.axis_index("s") * 2 + lax.axis_index("c")
    base = wid * chunk
    pltpu.sync_copy(y_ref.at[pl.ds(base, chunk)], y_v)
    for c in range(chunk // 16):
        row = base + c * 16 + lax.iota(jnp.int32, 16)
        idx_v[pl.ds(c * 16, 16)] = row * nclass + y_v[pl.ds(c * 16, 16)]
    pltpu.async_copy(flat_ref.at[idx_v], t_v, sem).wait()
    pltpu.sync_copy(t_v, t_ref.at[pl.ds(base, chunk)])


def _sc_gather(y_hat, y):
    b, n = y_hat.shape
    chunk = b // 32
    mesh = plsc.VectorSubcoreMesh(core_axis_name="c", subcore_axis_name="s")
    kfn = functools.partial(
        pl.kernel,
        mesh=mesh,
        out_type=jax.ShapeDtypeStruct((b,), jnp.float32),
        scratch_types=[
            pltpu.VMEM((chunk,), jnp.int32),
            pltpu.VMEM((chunk,), jnp.int32),
            pltpu.VMEM((chunk,), jnp.float32),
            pltpu.SemaphoreType.DMA,
        ],
    )(functools.partial(_sc_gather_body, n, chunk))
    return kfn(y_hat.reshape(b * n), y)


def _tc_body(t_ref, x_ref, out_ref, acc_ref, *, nrb, batch):
    i = pl.program_id(0)

    @pl.when(i == 0)
    def _init():
        acc_ref[...] = jnp.zeros_like(acc_ref)

    x = x_ref[...]
    e = jnp.exp2(x * _C1 - _C1)
    s = jnp.sum(e, axis=1, keepdims=True)

    t = t_ref[...]
    tm = _margined(t)
    e_t = jnp.exp2(t * _C1 - _C1)
    e_tm = jnp.exp2(tm * _C1 - _C1)
    s_mod = s - e_t + e_tm
    loss_rows = jnp.log(s_mod) + (_SCALE - _SCALE * tm)
    acc_ref[...] = acc_ref[...] + jnp.sum(loss_rows, axis=(0, 1), keepdims=True)

    @pl.when(i == nrb - 1)
    def _fin():
        out_ref[...] = acc_ref[...] / batch


def _tc_reduce(y_hat, t, rb, interpret=False):
    b, n = y_hat.shape
    nrb = b // rb
    out = pl.pallas_call(
        functools.partial(_tc_body, nrb=nrb, batch=b),
        grid=(nrb,),
        in_specs=[
            pl.BlockSpec((rb, 1), lambda i: (i, 0)),
            pl.BlockSpec((rb, n), lambda i: (i, 0)),
        ],
        out_specs=pl.BlockSpec((1, 1), lambda i: (0, 0)),
        out_shape=jax.ShapeDtypeStruct((1, 1), jnp.float32),
        scratch_shapes=[pltpu.VMEM((1, 1), jnp.float32)],
        interpret=interpret,
    )(t.reshape(b, 1), y_hat)
    return out[0, 0]


@jax.jit
def kernel(y_hat, y):
    t = _sc_gather(y_hat, y)
    return _tc_reduce(y_hat, t, 64)
